# Initial kernel scaffold; baseline (speedup 1.0000x reference)
import jax
import jax.numpy as jnp
from jax.experimental import pallas as pl


def kernel(entity_emb, relation_emb, adj_indices, adj_values, W1, W2):
    raise NotImplementedError("write your pallas kernel here")


# speedup vs baseline: 3.5464x; 3.5464x over previous
import functools

import jax
import jax.numpy as jnp
from jax import lax
from jax.experimental import pallas as pl
from jax.experimental.pallas import tpu as pltpu
from jax.experimental.pallas import tpu_sc as plsc

_NUM_NODES = 10000
_N_TOTAL = 10500
_D = 128
_E = 320000
_ALPHA = 0.2
_NPAD = 10624
_ROWS_PER_TILE = _NPAD // 16
_NW = 32
_EDGES_PER_TILE = _E // _NW
_C = 80
_NCHUNK = _EDGES_PER_TILE // _C
_MM_BLOCK = 1328
_VPE = _D // 16


def _sc_partial_segment_sum(x, src, dst, vals, zeros):
  mesh = plsc.VectorSubcoreMesh(core_axis_name="c", subcore_axis_name="s")

  @functools.partial(
      pl.kernel,
      mesh=mesh,
      out_type=jax.ShapeDtypeStruct((2, _NPAD, _D), jnp.float32),
      scratch_types=[
          pltpu.VMEM((_C,), jnp.int32),
          pltpu.VMEM((_C,), jnp.int32),
          pltpu.VMEM((_C + 16,), jnp.float32),
          pltpu.VMEM((_C, _D), jnp.float32),
          pltpu.VMEM_SHARED((_NPAD, _D), jnp.float32),
          pltpu.SemaphoreType.DMA,
      ],
  )
  def k(x_hbm, src_hbm, dst_hbm, vals_hbm, zeros_hbm, out_hbm,
        src_v, dst_v, vals_v, rows_v, acc_sh, sem):
    c = lax.axis_index("c")
    s = lax.axis_index("s")
    wid = s * 2 + c
    row0 = s * _ROWS_PER_TILE

    pltpu.sync_copy(zeros_hbm.at[pl.ds(row0, _ROWS_PER_TILE)],
                    acc_sh.at[pl.ds(row0, _ROWS_PER_TILE)])
    plsc.subcore_barrier()

    tile_base = wid * _EDGES_PER_TILE

    def chunk_body(j, carry):
      base = pl.multiple_of(tile_base + j * _C, 8)
      pltpu.sync_copy(src_hbm.at[pl.ds(base, _C)], src_v)
      pltpu.sync_copy(dst_hbm.at[pl.ds(base, _C)], dst_v)
      pltpu.sync_copy(vals_hbm.at[pl.ds(base, _C)], vals_v.at[pl.ds(0, _C)])
      pltpu.async_copy(x_hbm.at[src_v], rows_v, sem).wait()

      def edge_body(e, carry2):
        vv = vals_v[pl.ds(e, 16)][0]
        for d in range(_VPE):
          rows_v[e, pl.ds(d * 16, 16)] = rows_v[e, pl.ds(d * 16, 16)] * vv
        return carry2

      lax.fori_loop(0, _C, edge_body, 0)
      pltpu.sync_copy(rows_v, acc_sh.at[dst_v], add=True)
      return carry

    lax.fori_loop(0, _NCHUNK, chunk_body, 0)
    plsc.subcore_barrier()

    pltpu.sync_copy(acc_sh.at[pl.ds(row0, _ROWS_PER_TILE)],
                    out_hbm.at[c, pl.ds(row0, _ROWS_PER_TILE)])

  return k(x, src, dst, vals, zeros)


def _tc_layer(parts, w, normalize):
  def body(p_ref, w_ref, o_ref):
    sm = p_ref[0] + p_ref[1]
    y = jnp.dot(sm, w_ref[...], preferred_element_type=jnp.float32)
    y = jnp.where(y >= 0, y, _ALPHA * y)
    if normalize:
      nrm = jnp.sqrt(jnp.sum(y * y, axis=1, keepdims=True))
      y = y / jnp.maximum(nrm, 1e-12)
    o_ref[...] = y

  return pl.pallas_call(
      body,
      grid=(_NPAD // _MM_BLOCK,),
      in_specs=[
          pl.BlockSpec((2, _MM_BLOCK, _D), lambda i: (0, i, 0)),
          pl.BlockSpec((_D, _D), lambda i: (0, 0)),
      ],
      out_specs=pl.BlockSpec((_MM_BLOCK, _D), lambda i: (i, 0)),
      out_shape=jax.ShapeDtypeStruct((_NPAD, _D), jnp.float32),
  )(parts, w)


def kernel(entity_emb, relation_emb, adj_indices, adj_values, W1, W2):
  x = jnp.concatenate([entity_emb, relation_emb], axis=0)
  src = adj_indices[0]
  dst = adj_indices[1]
  zeros = jnp.zeros((_NPAD, _D), jnp.float32)

  p1 = _sc_partial_segment_sum(x, src, dst, adj_values, zeros)
  h1 = _tc_layer(p1, W1, normalize=False)
  p2 = _sc_partial_segment_sum(h1, src, dst, adj_values, zeros)
  out2 = _tc_layer(p2, W2, normalize=True)

  return (out2[:_NUM_NODES], out2[_NUM_NODES:_N_TOTAL])
